# baseline (speedup 1.0000x reference)
import functools

import jax
import jax.numpy as jnp
import numpy as np
from jax import lax
from jax.experimental import pallas as pl
from jax.experimental.pallas import tpu as pltpu
from jax.experimental.pallas import tpu_sc as plsc

_B = 16
_IMG = 512
_BASE = 16
_NBINS = 32
_WF = 32
_H2 = 16
_NC = 256
_K = 85
_N16 = 4 * _K
_N32 = _NC - _K
_MEAN = np.array([0.485, 0.456, 0.406], dtype=np.float32)
_STD = np.array([0.229, 0.224, 0.225], dtype=np.float32)

_HP = lax.Precision.HIGHEST

_D = 3 * _BASE * _BASE
_R16_N = _B * _N16
_RF_N = _B * _N32 * 4
_R32_N = _B * _N32
_R32_PAD = 43 * 64


def _hist_body(bins_ref, hist_ref):
    x = bins_ref[0]
    cols = [
        jnp.sum((x == b).astype(jnp.float32), axis=1, keepdims=True)
        for b in range(_NBINS)
    ]
    hist_ref[0] = jnp.concatenate(cols, axis=1)


def _hist_call(bins):
    return pl.pallas_call(
        _hist_body,
        grid=(_B,),
        in_specs=[pl.BlockSpec((1, _NC, 1024), lambda b: (b, 0, 0))],
        out_specs=pl.BlockSpec((1, _NC, _NBINS), lambda b: (b, 0, 0)),
        out_shape=jax.ShapeDtypeStruct((_B, _NC, _NBINS), jnp.float32),
    )(bins)


def _select_body(impr_ref, impc_ref, i16_ref, ifull_ref, i32s_ref, om_ref):
    b = pl.program_id(0)
    vr = impr_ref[0]
    vc = impc_ref[0]
    ii = lax.broadcasted_iota(jnp.int32, (_NC, _NC), 1)
    jj = lax.broadcasted_iota(jnp.int32, (_NC, _NC), 0)
    jlt = jj < ii
    eq = vc == vr
    rank_hi = jnp.sum(((vc > vr) | (eq & jlt)).astype(jnp.float32), axis=0,
                      keepdims=True)
    rank_lo = jnp.sum(((vc < vr) | (eq & jlt)).astype(jnp.float32), axis=0,
                      keepdims=True)
    refine = (rank_hi < float(_K)).astype(jnp.float32)
    keep = (rank_lo < float(_N32)).astype(jnp.float32)

    fio = lax.broadcasted_iota(jnp.int32, (_NC, 1024), 1)
    cio = lax.broadcasted_iota(jnp.int32, (_NC, 1024), 0)
    e2 = (((fio // 64) * 16 + (fio % 32) // 2) == cio).astype(jnp.float32)
    m1024 = jnp.dot(refine, e2, preferred_element_type=jnp.float32,
                    precision=_HP)

    g2 = lax.broadcasted_iota(jnp.int32, (1024, 1024), 0)
    f2 = lax.broadcasted_iota(jnp.int32, (1024, 1024), 1)
    u1024 = (g2 <= f2).astype(jnp.float32)
    pos16 = jnp.dot(m1024, u1024, preferred_element_type=jnp.float32,
                    precision=_HP)
    g3 = lax.broadcasted_iota(jnp.int32, (_NC, _NC), 0)
    f3 = lax.broadcasted_iota(jnp.int32, (_NC, _NC), 1)
    u256 = (g3 <= f3).astype(jnp.float32)
    pos32 = jnp.dot(keep, u256, preferred_element_type=jnp.float32,
                    precision=_HP)

    t1 = lax.broadcasted_iota(jnp.int32, (_N16, 1024), 0).astype(jnp.float32)
    o16 = ((pos16 == (t1 + 1.0)) & (m1024 > 0.0)).astype(jnp.float32)
    fcol = lax.broadcasted_iota(jnp.int32, (1024, 1), 0).astype(jnp.float32)
    idx16 = jnp.dot(o16, fcol, preferred_element_type=jnp.float32,
                    precision=_HP).astype(jnp.int32)
    t2 = lax.broadcasted_iota(jnp.int32, (_N32, _NC), 0).astype(jnp.float32)
    o32 = ((pos32 == (t2 + 1.0)) & (keep > 0.0)).astype(jnp.float32)
    ccol = lax.broadcasted_iota(jnp.int32, (_NC, 1), 0).astype(jnp.float32)
    idx32 = jnp.dot(o32, ccol, preferred_element_type=jnp.float32,
                    precision=_HP).astype(jnp.int32)

    i16_ref[0] = b * 1024 + idx16

    r, cc = idx32 // 16, idx32 % 16
    sub = lax.broadcasted_iota(jnp.int32, (1, 4), 1)
    ifull_ref[0] = (b * 1024 + (2 * r + sub // 2) * 32
                    + 2 * cc + sub % 2)

    i32s_ref[0] = b * _NC + idx32

    lmask = lax.broadcasted_iota(jnp.int32, (1, 512), 1)
    om_ref[0] = jnp.where(lmask == 0, -1.0,
                          jnp.where(lmask <= _N16, 1.0, 2.0))


def _select_call(imp):
    return pl.pallas_call(
        _select_body,
        grid=(_B,),
        in_specs=[
            pl.BlockSpec((1, 1, _NC), lambda b: (b, 0, 0)),
            pl.BlockSpec((1, _NC, 1), lambda b: (b, 0, 0)),
        ],
        out_specs=[
            pl.BlockSpec((1, _N16, 1), lambda b: (b, 0, 0)),
            pl.BlockSpec((1, _N32, 4), lambda b: (b, 0, 0)),
            pl.BlockSpec((1, _N32, 1), lambda b: (b, 0, 0)),
            pl.BlockSpec((1, 1, 512), lambda b: (b, 0, 0)),
        ],
        out_shape=[
            jax.ShapeDtypeStruct((_B, _N16, 1), jnp.int32),
            jax.ShapeDtypeStruct((_B, _N32, 4), jnp.int32),
            jax.ShapeDtypeStruct((_B, _N32, 1), jnp.int32),
            jax.ShapeDtypeStruct((_B, 1, 512), jnp.float32),
        ],
    )(imp.reshape(_B, 1, _NC), imp.reshape(_B, _NC, 1))


def _resize_body(img_ref, small_ref):
    def tri(d):
        w = jnp.where((d == 0) | (d == 1), 3.0, 0.0)
        return w + jnp.where((d == -1) | (d == 2), 1.0, 0.0)

    io2 = lax.broadcasted_iota(jnp.int32, (256, 512), 0)
    jo2 = lax.broadcasted_iota(jnp.int32, (256, 512), 1)
    den_r = jnp.where((io2 == 0) | (io2 == 255), 7.0, 8.0)
    w_r = tri(jo2 - 2 * io2) / den_r
    io3 = lax.broadcasted_iota(jnp.int32, (512, 256), 1)
    jo3 = lax.broadcasted_iota(jnp.int32, (512, 256), 0)
    den_c = jnp.where((io3 == 0) | (io3 == 255), 7.0, 8.0)
    w_c = tri(jo3 - 2 * io3) / den_c
    for c in range(3):
        t = jnp.dot(w_r, img_ref[0, c], preferred_element_type=jnp.float32,
                    precision=_HP)
        small_ref[0, c] = jnp.dot(t, w_c, preferred_element_type=jnp.float32,
                                  precision=_HP)


def _resize_call(images):
    return pl.pallas_call(
        _resize_body,
        grid=(_B,),
        in_specs=[pl.BlockSpec((1, 3, _IMG, _IMG), lambda b: (b, 0, 0, 0))],
        out_specs=pl.BlockSpec((1, 3, 256, 256), lambda b: (b, 0, 0, 0)),
        out_shape=jax.ShapeDtypeStruct((_B, 3, 256, 256), jnp.float32),
    )(images)


def _sc_gather(fine_t, coarse_t, i16, ifull, i32s):
    mesh = plsc.VectorSubcoreMesh(core_axis_name="c", subcore_axis_name="s")

    @functools.partial(
        pl.kernel,
        mesh=mesh,
        out_type=[
            jax.ShapeDtypeStruct((_R16_N, _D), jnp.float32),
            jax.ShapeDtypeStruct((_RF_N, _D), jnp.float32),
            jax.ShapeDtypeStruct((_R32_PAD, _D), jnp.float32),
        ],
        scratch_types=[
            pltpu.VMEM((64,), jnp.int32),
            pltpu.VMEM((64, _D), jnp.float32),
            pltpu.SemaphoreType.DMA,
        ],
    )
    def k(finet, coarset, i16h, ifullh, i32h, o16, ofull, o32, idxv, buf, sem):
        wid = lax.axis_index("s") * 2 + lax.axis_index("c")

        def make_phase(idx_hbm, table, out_hbm, nchunk):
            def body(js, carry):
                cid = js * 32 + wid

                @pl.when(cid < nchunk)
                def _():
                    base = cid * 64
                    pltpu.sync_copy(idx_hbm.at[pl.ds(base, 64)], idxv)
                    pltpu.async_copy(table.at[idxv], buf, sem).wait()
                    pltpu.sync_copy(buf, out_hbm.at[pl.ds(base, 64)])

                return carry

            lax.fori_loop(0, (nchunk + 31) // 32, body, 0)

        make_phase(i16h, finet, o16, _R16_N // 64)
        make_phase(ifullh, finet, ofull, _RF_N // 64)
        make_phase(i32h, coarset, o32, _R32_PAD // 64)

    return k(fine_t, coarse_t, i16, ifull, i32s)


def kernel(images):
    imgs01 = (images * jnp.asarray(_STD).reshape(1, 3, 1, 1)
              + jnp.asarray(_MEAN).reshape(1, 3, 1, 1))
    gray = imgs01.mean(axis=1)
    g = (gray.reshape(_B, _H2, 32, _H2, 32).transpose(0, 1, 3, 2, 4)
         .reshape(_B, _NC, 1024))
    bins = jnp.clip((g * _NBINS).astype(jnp.int32), 0, _NBINS - 1)

    hist = _hist_call(bins)

    p = hist.reshape(_B, _H2, _H2, _NBINS) / float(32 * 32)
    ent = -jnp.sum(jnp.where(p > 0, p * jnp.log2(jnp.maximum(p, 1e-12)), 0.0),
                   axis=-1)
    imp = ent.reshape(_B, _NC)

    i16, ifull, i32s, om = _select_call(imp)
    small = _resize_call(images)

    fine_t = (images.reshape(_B, 3, _WF, _BASE, _WF, _BASE)
              .transpose(0, 2, 4, 1, 3, 5).reshape(_B * 1024, _D))
    coarse_t = (small.reshape(_B, 3, _H2, _BASE, _H2, _BASE)
                .transpose(0, 2, 4, 1, 3, 5).reshape(_B * _NC, _D))

    i32s_pad = jnp.concatenate(
        [i32s.reshape(-1), jnp.zeros((_R32_PAD - _R32_N,), jnp.int32)])
    o16, ofull, o32 = _sc_gather(
        fine_t, coarse_t, i16.reshape(-1), ifull.reshape(-1), i32s_pad)

    resized16 = o16.reshape(-1, 3, _BASE, _BASE)
    full32 = ofull.reshape(-1, 4, 3, _BASE, _BASE)
    resized32 = o32[:_R32_N].reshape(-1, 3, _BASE, _BASE)
    output_mask = om.reshape(-1)
    seqlens = jnp.full((_B,), 1 + _N16 + _N32, jnp.int32)
    return (resized16, resized32, full32, output_mask, seqlens)


# speedup vs baseline: 1.2960x; 1.2960x over previous
import functools

import jax
import jax.numpy as jnp
import numpy as np
from jax import lax
from jax.experimental import pallas as pl
from jax.experimental.pallas import tpu as pltpu
from jax.experimental.pallas import tpu_sc as plsc

_B = 16
_IMG = 512
_BASE = 16
_NBINS = 32
_WF = 32
_H2 = 16
_NC = 256
_K = 85
_N16 = 4 * _K
_N32 = _NC - _K
_MEAN = np.array([0.485, 0.456, 0.406], dtype=np.float32)
_STD = np.array([0.229, 0.224, 0.225], dtype=np.float32)

_HP = lax.Precision.HIGHEST

_D = 3 * _BASE * _BASE
_R16_N = _B * _N16
_RF_N = _B * _N32 * 4
_R32_N = _B * _N32
_R32_PAD = 43 * 64


def _hist_body(bins_ref, hist_ref):
    x = bins_ref[0]
    cols = [
        jnp.sum((x == b).astype(jnp.float32), axis=1, keepdims=True)
        for b in range(_NBINS)
    ]
    hist_ref[0] = jnp.concatenate(cols, axis=1)


def _hist_call(bins):
    return pl.pallas_call(
        _hist_body,
        grid=(_B,),
        in_specs=[pl.BlockSpec((1, _NC, 1024), lambda b: (b, 0, 0))],
        out_specs=pl.BlockSpec((1, _NC, _NBINS), lambda b: (b, 0, 0)),
        out_shape=jax.ShapeDtypeStruct((_B, _NC, _NBINS), jnp.float32),
    )(bins)


def _select_body(impr_ref, impc_ref, i16_ref, ifull_ref, i32s_ref, om_ref):
    b = pl.program_id(0)
    vr = impr_ref[0]
    vc = impc_ref[0]
    ii = lax.broadcasted_iota(jnp.int32, (_NC, _NC), 1)
    jj = lax.broadcasted_iota(jnp.int32, (_NC, _NC), 0)
    jlt = jj < ii
    eq = vc == vr
    rank_hi = jnp.sum(((vc > vr) | (eq & jlt)).astype(jnp.float32), axis=0,
                      keepdims=True)
    rank_lo = jnp.sum(((vc < vr) | (eq & jlt)).astype(jnp.float32), axis=0,
                      keepdims=True)
    refine = (rank_hi < float(_K)).astype(jnp.float32)
    keep = (rank_lo < float(_N32)).astype(jnp.float32)

    fio = lax.broadcasted_iota(jnp.int32, (_NC, 1024), 1)
    cio = lax.broadcasted_iota(jnp.int32, (_NC, 1024), 0)
    e2 = (((fio // 64) * 16 + (fio % 32) // 2) == cio).astype(jnp.float32)
    m1024 = jnp.dot(refine, e2, preferred_element_type=jnp.float32,
                    precision=_HP)

    g2 = lax.broadcasted_iota(jnp.int32, (1024, 1024), 0)
    f2 = lax.broadcasted_iota(jnp.int32, (1024, 1024), 1)
    u1024 = (g2 <= f2).astype(jnp.float32)
    pos16 = jnp.dot(m1024, u1024, preferred_element_type=jnp.float32,
                    precision=_HP)
    g3 = lax.broadcasted_iota(jnp.int32, (_NC, _NC), 0)
    f3 = lax.broadcasted_iota(jnp.int32, (_NC, _NC), 1)
    u256 = (g3 <= f3).astype(jnp.float32)
    pos32 = jnp.dot(keep, u256, preferred_element_type=jnp.float32,
                    precision=_HP)

    t1 = lax.broadcasted_iota(jnp.int32, (_N16, 1024), 0).astype(jnp.float32)
    o16 = ((pos16 == (t1 + 1.0)) & (m1024 > 0.0)).astype(jnp.float32)
    fcol = lax.broadcasted_iota(jnp.int32, (1024, 1), 0).astype(jnp.float32)
    idx16 = jnp.dot(o16, fcol, preferred_element_type=jnp.float32,
                    precision=_HP).astype(jnp.int32)
    t2 = lax.broadcasted_iota(jnp.int32, (_N32, _NC), 0).astype(jnp.float32)
    o32 = ((pos32 == (t2 + 1.0)) & (keep > 0.0)).astype(jnp.float32)
    ccol = lax.broadcasted_iota(jnp.int32, (_NC, 1), 0).astype(jnp.float32)
    idx32 = jnp.dot(o32, ccol, preferred_element_type=jnp.float32,
                    precision=_HP).astype(jnp.int32)

    i16_ref[0] = b * 1024 + idx16

    r, cc = idx32 // 16, idx32 % 16
    sub = lax.broadcasted_iota(jnp.int32, (1, 4), 1)
    ifull_ref[0] = (b * 1024 + (2 * r + sub // 2) * 32
                    + 2 * cc + sub % 2)

    i32s_ref[0] = b * _NC + idx32

    lmask = lax.broadcasted_iota(jnp.int32, (1, 512), 1)
    om_ref[0] = jnp.where(lmask == 0, -1.0,
                          jnp.where(lmask <= _N16, 1.0, 2.0))


def _select_call(imp):
    return pl.pallas_call(
        _select_body,
        grid=(_B,),
        in_specs=[
            pl.BlockSpec((1, 1, _NC), lambda b: (b, 0, 0)),
            pl.BlockSpec((1, _NC, 1), lambda b: (b, 0, 0)),
        ],
        out_specs=[
            pl.BlockSpec((1, _N16, 1), lambda b: (b, 0, 0)),
            pl.BlockSpec((1, _N32, 4), lambda b: (b, 0, 0)),
            pl.BlockSpec((1, _N32, 1), lambda b: (b, 0, 0)),
            pl.BlockSpec((1, 1, 512), lambda b: (b, 0, 0)),
        ],
        out_shape=[
            jax.ShapeDtypeStruct((_B, _N16, 1), jnp.int32),
            jax.ShapeDtypeStruct((_B, _N32, 4), jnp.int32),
            jax.ShapeDtypeStruct((_B, _N32, 1), jnp.int32),
            jax.ShapeDtypeStruct((_B, 1, 512), jnp.float32),
        ],
    )(imp.reshape(_B, 1, _NC), imp.reshape(_B, _NC, 1))


def _prep_body(img_ref, fine_ref, coarse_ref):
    def tri(d):
        w = jnp.where((d == 0) | (d == 1), 3.0, 0.0)
        return w + jnp.where((d == -1) | (d == 2), 1.0, 0.0)

    io2 = lax.broadcasted_iota(jnp.int32, (256, 512), 0)
    jo2 = lax.broadcasted_iota(jnp.int32, (256, 512), 1)
    den_r = jnp.where((io2 == 0) | (io2 == 255), 7.0, 8.0)
    w_r = tri(jo2 - 2 * io2) / den_r
    io3 = lax.broadcasted_iota(jnp.int32, (512, 256), 1)
    jo3 = lax.broadcasted_iota(jnp.int32, (512, 256), 0)
    den_c = jnp.where((io3 == 0) | (io3 == 255), 7.0, 8.0)
    w_c = tri(jo3 - 2 * io3) / den_c

    def patchify(x, n):
        t = x.reshape(n, _BASE, n, _BASE).transpose(0, 2, 1, 3)
        return t.reshape(n * n, 256)

    for c in range(3):
        fine_ref[0, :, 256 * c:256 * (c + 1)] = patchify(img_ref[0, c], _WF)
        t = jnp.dot(w_r, img_ref[0, c], preferred_element_type=jnp.float32,
                    precision=_HP)
        sm = jnp.dot(t, w_c, preferred_element_type=jnp.float32,
                     precision=_HP)
        coarse_ref[0, :, 256 * c:256 * (c + 1)] = patchify(sm, _H2)


def _prep_call(images):
    return pl.pallas_call(
        _prep_body,
        grid=(_B,),
        in_specs=[pl.BlockSpec((1, 3, _IMG, _IMG), lambda b: (b, 0, 0, 0))],
        out_specs=[
            pl.BlockSpec((1, 1024, _D), lambda b: (b, 0, 0)),
            pl.BlockSpec((1, _NC, _D), lambda b: (b, 0, 0)),
        ],
        out_shape=[
            jax.ShapeDtypeStruct((_B, 1024, _D), jnp.float32),
            jax.ShapeDtypeStruct((_B, _NC, _D), jnp.float32),
        ],
    )(images)


def _sc_gather(fine_t, coarse_t, i16, ifull, i32s):
    mesh = plsc.VectorSubcoreMesh(core_axis_name="c", subcore_axis_name="s")

    @functools.partial(
        pl.kernel,
        mesh=mesh,
        out_type=[
            jax.ShapeDtypeStruct((_R16_N, _D), jnp.float32),
            jax.ShapeDtypeStruct((_RF_N, _D), jnp.float32),
            jax.ShapeDtypeStruct((_R32_PAD, _D), jnp.float32),
        ],
        scratch_types=[
            pltpu.VMEM((64,), jnp.int32),
            pltpu.VMEM((64, _D), jnp.float32),
            pltpu.SemaphoreType.DMA,
        ],
    )
    def k(finet, coarset, i16h, ifullh, i32h, o16, ofull, o32, idxv, buf, sem):
        wid = lax.axis_index("s") * 2 + lax.axis_index("c")

        def make_phase(idx_hbm, table, out_hbm, nchunk):
            def body(js, carry):
                cid = js * 32 + wid

                @pl.when(cid < nchunk)
                def _():
                    base = cid * 64
                    pltpu.sync_copy(idx_hbm.at[pl.ds(base, 64)], idxv)
                    pltpu.async_copy(table.at[idxv], buf, sem).wait()
                    pltpu.sync_copy(buf, out_hbm.at[pl.ds(base, 64)])

                return carry

            lax.fori_loop(0, (nchunk + 31) // 32, body, 0)

        make_phase(i16h, finet, o16, _R16_N // 64)
        make_phase(ifullh, finet, ofull, _RF_N // 64)
        make_phase(i32h, coarset, o32, _R32_PAD // 64)

    return k(fine_t, coarse_t, i16, ifull, i32s)


def kernel(images):
    imgs01 = (images * jnp.asarray(_STD).reshape(1, 3, 1, 1)
              + jnp.asarray(_MEAN).reshape(1, 3, 1, 1))
    gray = imgs01.mean(axis=1)
    g = (gray.reshape(_B, _H2, 32, _H2, 32).transpose(0, 1, 3, 2, 4)
         .reshape(_B, _NC, 1024))
    bins = jnp.clip((g * _NBINS).astype(jnp.int32), 0, _NBINS - 1)

    hist = _hist_call(bins)

    p = hist.reshape(_B, _H2, _H2, _NBINS) / float(32 * 32)
    ent = -jnp.sum(jnp.where(p > 0, p * jnp.log2(jnp.maximum(p, 1e-12)), 0.0),
                   axis=-1)
    imp = ent.reshape(_B, _NC)

    i16, ifull, i32s, om = _select_call(imp)
    fine_t4, coarse_t4 = _prep_call(images)
    fine_t = fine_t4.reshape(_B * 1024, _D)
    coarse_t = coarse_t4.reshape(_B * _NC, _D)

    i32s_pad = jnp.concatenate(
        [i32s.reshape(-1), jnp.zeros((_R32_PAD - _R32_N,), jnp.int32)])
    o16, ofull, o32 = _sc_gather(
        fine_t, coarse_t, i16.reshape(-1), ifull.reshape(-1), i32s_pad)

    resized16 = o16.reshape(-1, 3, _BASE, _BASE)
    full32 = ofull.reshape(-1, 4, 3, _BASE, _BASE)
    resized32 = o32[:_R32_N].reshape(-1, 3, _BASE, _BASE)
    output_mask = om.reshape(-1)
    seqlens = jnp.full((_B,), 1 + _N16 + _N32, jnp.int32)
    return (resized16, resized32, full32, output_mask, seqlens)


# speedup vs baseline: 1.3619x; 1.0508x over previous
import functools

import jax
import jax.numpy as jnp
import numpy as np
from jax import lax
from jax.experimental import pallas as pl
from jax.experimental.pallas import tpu as pltpu
from jax.experimental.pallas import tpu_sc as plsc

_B = 16
_IMG = 512
_BASE = 16
_NBINS = 32
_WF = 32
_H2 = 16
_NC = 256
_K = 85
_N16 = 4 * _K
_N32 = _NC - _K
_MEAN = np.array([0.485, 0.456, 0.406], dtype=np.float32)
_STD = np.array([0.229, 0.224, 0.225], dtype=np.float32)

_HP = lax.Precision.HIGHEST

_D = 3 * _BASE * _BASE
_R16_N = _B * _N16
_RF_N = _B * _N32 * 4
_R32_N = _B * _N32
_R32_PAD = 43 * 64


def _hist_body(bins_ref, hist_ref):
    x = bins_ref[0]
    jio = lax.broadcasted_iota(jnp.int32, (512, _H2), 0)
    cio = lax.broadcasted_iota(jnp.int32, (512, _H2), 1)
    colsum = (jio // 32 == cio).astype(jnp.float32)
    planes = []
    for b in range(_NBINS):
        m = (x == b).astype(jnp.float32)
        rowsum = jnp.sum(m.reshape(_H2, 32, 512), axis=1)
        patch = jnp.dot(rowsum, colsum, preferred_element_type=jnp.float32,
                        precision=_HP)
        planes.append(patch[:, :, None])
    hist_ref[0] = jnp.concatenate(planes, axis=2)


def _hist_call(bins):
    return pl.pallas_call(
        _hist_body,
        grid=(_B,),
        in_specs=[pl.BlockSpec((1, _IMG, _IMG), lambda b: (b, 0, 0))],
        out_specs=pl.BlockSpec((1, _H2, _H2, _NBINS), lambda b: (b, 0, 0, 0)),
        out_shape=jax.ShapeDtypeStruct((_B, _H2, _H2, _NBINS), jnp.float32),
    )(bins)


def _select_body(impr_ref, impc_ref, i16_ref, ifull_ref, i32s_ref, om_ref):
    b = pl.program_id(0)
    vr = impr_ref[0]
    vc = impc_ref[0]
    ii = lax.broadcasted_iota(jnp.int32, (_NC, _NC), 1)
    jj = lax.broadcasted_iota(jnp.int32, (_NC, _NC), 0)
    jlt = jj < ii
    eq = vc == vr
    rank_hi = jnp.sum(((vc > vr) | (eq & jlt)).astype(jnp.float32), axis=0,
                      keepdims=True)
    rank_lo = jnp.sum(((vc < vr) | (eq & jlt)).astype(jnp.float32), axis=0,
                      keepdims=True)
    refine = (rank_hi < float(_K)).astype(jnp.float32)
    keep = (rank_lo < float(_N32)).astype(jnp.float32)

    fio = lax.broadcasted_iota(jnp.int32, (_NC, 1024), 1)
    cio = lax.broadcasted_iota(jnp.int32, (_NC, 1024), 0)
    e2 = (((fio // 64) * 16 + (fio % 32) // 2) == cio).astype(jnp.float32)
    m1024 = jnp.dot(refine, e2, preferred_element_type=jnp.float32,
                    precision=_HP)

    g2 = lax.broadcasted_iota(jnp.int32, (1024, 1024), 0)
    f2 = lax.broadcasted_iota(jnp.int32, (1024, 1024), 1)
    u1024 = (g2 <= f2).astype(jnp.float32)
    pos16 = jnp.dot(m1024, u1024, preferred_element_type=jnp.float32,
                    precision=_HP)
    g3 = lax.broadcasted_iota(jnp.int32, (_NC, _NC), 0)
    f3 = lax.broadcasted_iota(jnp.int32, (_NC, _NC), 1)
    u256 = (g3 <= f3).astype(jnp.float32)
    pos32 = jnp.dot(keep, u256, preferred_element_type=jnp.float32,
                    precision=_HP)

    t1 = lax.broadcasted_iota(jnp.int32, (_N16, 1024), 0).astype(jnp.float32)
    o16 = ((pos16 == (t1 + 1.0)) & (m1024 > 0.0)).astype(jnp.float32)
    fcol = lax.broadcasted_iota(jnp.int32, (1024, 1), 0).astype(jnp.float32)
    idx16 = jnp.dot(o16, fcol, preferred_element_type=jnp.float32,
                    precision=_HP).astype(jnp.int32)
    t2 = lax.broadcasted_iota(jnp.int32, (_N32, _NC), 0).astype(jnp.float32)
    o32 = ((pos32 == (t2 + 1.0)) & (keep > 0.0)).astype(jnp.float32)
    ccol = lax.broadcasted_iota(jnp.int32, (_NC, 1), 0).astype(jnp.float32)
    idx32 = jnp.dot(o32, ccol, preferred_element_type=jnp.float32,
                    precision=_HP).astype(jnp.int32)

    i16_ref[0] = b * 1024 + idx16

    r, cc = idx32 // 16, idx32 % 16
    sub = lax.broadcasted_iota(jnp.int32, (1, 4), 1)
    ifull_ref[0] = (b * 1024 + (2 * r + sub // 2) * 32
                    + 2 * cc + sub % 2)

    i32s_ref[0] = b * _NC + idx32

    lmask = lax.broadcasted_iota(jnp.int32, (1, 512), 1)
    om_ref[0] = jnp.where(lmask == 0, -1.0,
                          jnp.where(lmask <= _N16, 1.0, 2.0))


def _select_call(imp):
    return pl.pallas_call(
        _select_body,
        grid=(_B,),
        in_specs=[
            pl.BlockSpec((1, 1, _NC), lambda b: (b, 0, 0)),
            pl.BlockSpec((1, _NC, 1), lambda b: (b, 0, 0)),
        ],
        out_specs=[
            pl.BlockSpec((1, _N16, 1), lambda b: (b, 0, 0)),
            pl.BlockSpec((1, _N32, 4), lambda b: (b, 0, 0)),
            pl.BlockSpec((1, _N32, 1), lambda b: (b, 0, 0)),
            pl.BlockSpec((1, 1, 512), lambda b: (b, 0, 0)),
        ],
        out_shape=[
            jax.ShapeDtypeStruct((_B, _N16, 1), jnp.int32),
            jax.ShapeDtypeStruct((_B, _N32, 4), jnp.int32),
            jax.ShapeDtypeStruct((_B, _N32, 1), jnp.int32),
            jax.ShapeDtypeStruct((_B, 1, 512), jnp.float32),
        ],
    )(imp.reshape(_B, 1, _NC), imp.reshape(_B, _NC, 1))


def _prep_body(img_ref, fine_ref, coarse_ref):
    def tri(d):
        w = jnp.where((d == 0) | (d == 1), 3.0, 0.0)
        return w + jnp.where((d == -1) | (d == 2), 1.0, 0.0)

    io2 = lax.broadcasted_iota(jnp.int32, (256, 512), 0)
    jo2 = lax.broadcasted_iota(jnp.int32, (256, 512), 1)
    den_r = jnp.where((io2 == 0) | (io2 == 255), 7.0, 8.0)
    w_r = tri(jo2 - 2 * io2) / den_r
    io3 = lax.broadcasted_iota(jnp.int32, (512, 256), 1)
    jo3 = lax.broadcasted_iota(jnp.int32, (512, 256), 0)
    den_c = jnp.where((io3 == 0) | (io3 == 255), 7.0, 8.0)
    w_c = tri(jo3 - 2 * io3) / den_c

    def patchify(x, n):
        t = x.reshape(n, _BASE, n, _BASE).transpose(0, 2, 1, 3)
        return t.reshape(n * n, 256)

    for c in range(3):
        fine_ref[0, :, 256 * c:256 * (c + 1)] = patchify(img_ref[0, c], _WF)
        t = jnp.dot(w_r, img_ref[0, c], preferred_element_type=jnp.float32,
                    precision=_HP)
        sm = jnp.dot(t, w_c, preferred_element_type=jnp.float32,
                     precision=_HP)
        coarse_ref[0, :, 256 * c:256 * (c + 1)] = patchify(sm, _H2)


def _prep_call(images):
    return pl.pallas_call(
        _prep_body,
        grid=(_B,),
        in_specs=[pl.BlockSpec((1, 3, _IMG, _IMG), lambda b: (b, 0, 0, 0))],
        out_specs=[
            pl.BlockSpec((1, 1024, _D), lambda b: (b, 0, 0)),
            pl.BlockSpec((1, _NC, _D), lambda b: (b, 0, 0)),
        ],
        out_shape=[
            jax.ShapeDtypeStruct((_B, 1024, _D), jnp.float32),
            jax.ShapeDtypeStruct((_B, _NC, _D), jnp.float32),
        ],
    )(images)


def _sc_gather(fine_t, coarse_t, i16, ifull, i32s):
    mesh = plsc.VectorSubcoreMesh(core_axis_name="c", subcore_axis_name="s")

    @functools.partial(
        pl.kernel,
        mesh=mesh,
        out_type=[
            jax.ShapeDtypeStruct((_R16_N, _D), jnp.float32),
            jax.ShapeDtypeStruct((_RF_N, _D), jnp.float32),
            jax.ShapeDtypeStruct((_R32_PAD, _D), jnp.float32),
        ],
        scratch_types=[
            pltpu.VMEM((64,), jnp.int32),
            pltpu.VMEM((64, _D), jnp.float32),
            pltpu.SemaphoreType.DMA,
        ],
    )
    def k(finet, coarset, i16h, ifullh, i32h, o16, ofull, o32, idxv, buf, sem):
        wid = lax.axis_index("s") * 2 + lax.axis_index("c")

        def make_phase(idx_hbm, table, out_hbm, nchunk):
            def body(js, carry):
                cid = js * 32 + wid

                @pl.when(cid < nchunk)
                def _():
                    base = cid * 64
                    pltpu.sync_copy(idx_hbm.at[pl.ds(base, 64)], idxv)
                    pltpu.async_copy(table.at[idxv], buf, sem).wait()
                    pltpu.sync_copy(buf, out_hbm.at[pl.ds(base, 64)])

                return carry

            lax.fori_loop(0, (nchunk + 31) // 32, body, 0)

        make_phase(i16h, finet, o16, _R16_N // 64)
        make_phase(ifullh, finet, ofull, _RF_N // 64)
        make_phase(i32h, coarset, o32, _R32_PAD // 64)

    return k(fine_t, coarse_t, i16, ifull, i32s)


def kernel(images):
    imgs01 = (images * jnp.asarray(_STD).reshape(1, 3, 1, 1)
              + jnp.asarray(_MEAN).reshape(1, 3, 1, 1))
    gray = imgs01.mean(axis=1)
    bins = jnp.clip((gray * _NBINS).astype(jnp.int32), 0, _NBINS - 1)

    hist = _hist_call(bins)

    p = hist / float(32 * 32)
    ent = -jnp.sum(jnp.where(p > 0, p * jnp.log2(jnp.maximum(p, 1e-12)), 0.0),
                   axis=-1)
    imp = ent.reshape(_B, _NC)

    i16, ifull, i32s, om = _select_call(imp)
    fine_t4, coarse_t4 = _prep_call(images)
    fine_t = fine_t4.reshape(_B * 1024, _D)
    coarse_t = coarse_t4.reshape(_B * _NC, _D)

    i32s_pad = jnp.concatenate(
        [i32s.reshape(-1), jnp.zeros((_R32_PAD - _R32_N,), jnp.int32)])
    o16, ofull, o32 = _sc_gather(
        fine_t, coarse_t, i16.reshape(-1), ifull.reshape(-1), i32s_pad)

    resized16 = o16.reshape(-1, 3, _BASE, _BASE)
    full32 = ofull.reshape(-1, 4, 3, _BASE, _BASE)
    resized32 = o32[:_R32_N].reshape(-1, 3, _BASE, _BASE)
    output_mask = om.reshape(-1)
    seqlens = jnp.full((_B,), 1 + _N16 + _N32, jnp.int32)
    return (resized16, resized32, full32, output_mask, seqlens)


# speedup vs baseline: 1.4230x; 1.0449x over previous
import functools

import jax
import jax.numpy as jnp
import numpy as np
from jax import lax
from jax.experimental import pallas as pl
from jax.experimental.pallas import tpu as pltpu
from jax.experimental.pallas import tpu_sc as plsc

_B = 16
_IMG = 512
_BASE = 16
_NBINS = 32
_WF = 32
_H2 = 16
_NC = 256
_K = 85
_N16 = 4 * _K
_N32 = _NC - _K
_MEAN = np.array([0.485, 0.456, 0.406], dtype=np.float32)
_STD = np.array([0.229, 0.224, 0.225], dtype=np.float32)

_HP = lax.Precision.HIGHEST

_SEG16_N = _B * _N16 * 48
_SEGF_N = _B * _N32 * 192
_SEG32_N = _B * _N32 * 48
_SEG32_PAD = 129 * 1024


def _hist_body(bins_ref, hist_ref):
    x = bins_ref[0]
    jio = lax.broadcasted_iota(jnp.int32, (512, _H2), 0)
    cio = lax.broadcasted_iota(jnp.int32, (512, _H2), 1)
    colsum = (jio // 32 == cio).astype(jnp.float32)
    planes = []
    for b in range(_NBINS):
        m = (x == b).astype(jnp.float32)
        rowsum = jnp.sum(m.reshape(_H2, 32, 512), axis=1)
        patch = jnp.dot(rowsum, colsum, preferred_element_type=jnp.float32,
                        precision=_HP)
        planes.append(patch[:, :, None])
    hist_ref[0] = jnp.concatenate(planes, axis=2)


def _hist_call(bins):
    return pl.pallas_call(
        _hist_body,
        grid=(_B,),
        in_specs=[pl.BlockSpec((1, _IMG, _IMG), lambda b: (b, 0, 0))],
        out_specs=pl.BlockSpec((1, _H2, _H2, _NBINS), lambda b: (b, 0, 0, 0)),
        out_shape=jax.ShapeDtypeStruct((_B, _H2, _H2, _NBINS), jnp.float32),
    )(bins)


def _select_body(impr_ref, impc_ref, i16_ref, ifull_ref, i32s_ref, om_ref):
    b = pl.program_id(0)
    vr = impr_ref[0]
    vc = impc_ref[0]
    ii = lax.broadcasted_iota(jnp.int32, (_NC, _NC), 1)
    jj = lax.broadcasted_iota(jnp.int32, (_NC, _NC), 0)
    jlt = jj < ii
    eq = vc == vr
    rank_hi = jnp.sum(((vc > vr) | (eq & jlt)).astype(jnp.float32), axis=0,
                      keepdims=True)
    rank_lo = jnp.sum(((vc < vr) | (eq & jlt)).astype(jnp.float32), axis=0,
                      keepdims=True)
    refine = (rank_hi < float(_K)).astype(jnp.float32)
    keep = (rank_lo < float(_N32)).astype(jnp.float32)

    fio = lax.broadcasted_iota(jnp.int32, (_NC, 1024), 1)
    cio = lax.broadcasted_iota(jnp.int32, (_NC, 1024), 0)
    e2 = (((fio // 64) * 16 + (fio % 32) // 2) == cio).astype(jnp.float32)
    m1024 = jnp.dot(refine, e2, preferred_element_type=jnp.float32,
                    precision=_HP)

    g2 = lax.broadcasted_iota(jnp.int32, (1024, 1024), 0)
    f2 = lax.broadcasted_iota(jnp.int32, (1024, 1024), 1)
    u1024 = (g2 <= f2).astype(jnp.float32)
    pos16 = jnp.dot(m1024, u1024, preferred_element_type=jnp.float32,
                    precision=_HP)
    g3 = lax.broadcasted_iota(jnp.int32, (_NC, _NC), 0)
    f3 = lax.broadcasted_iota(jnp.int32, (_NC, _NC), 1)
    u256 = (g3 <= f3).astype(jnp.float32)
    pos32 = jnp.dot(keep, u256, preferred_element_type=jnp.float32,
                    precision=_HP)

    t1 = lax.broadcasted_iota(jnp.int32, (_N16, 1024), 0).astype(jnp.float32)
    o16 = ((pos16 == (t1 + 1.0)) & (m1024 > 0.0)).astype(jnp.float32)
    fcol = lax.broadcasted_iota(jnp.int32, (1024, 1), 0).astype(jnp.float32)
    idx16 = jnp.dot(o16, fcol, preferred_element_type=jnp.float32,
                    precision=_HP).astype(jnp.int32)
    t2 = lax.broadcasted_iota(jnp.int32, (_N32, _NC), 0).astype(jnp.float32)
    o32 = ((pos32 == (t2 + 1.0)) & (keep > 0.0)).astype(jnp.float32)
    ccol = lax.broadcasted_iota(jnp.int32, (_NC, 1), 0).astype(jnp.float32)
    idx32 = jnp.dot(o32, ccol, preferred_element_type=jnp.float32,
                    precision=_HP).astype(jnp.int32)

    cy = lax.broadcasted_iota(jnp.int32, (1, 48), 1)
    ch, y = cy // 16, cy % 16
    i16_ref[0] = (b * 49152 + ch * 16384 + (idx16 // 32) * 512 + y * 32
                  + idx16 % 32)

    r, cc = idx32 // 16, idx32 % 16
    q = lax.broadcasted_iota(jnp.int32, (1, 192), 1)
    sub, chf, yf = q // 48, (q % 48) // 16, q % 16
    ifull_ref[0] = (b * 49152 + chf * 16384 + (2 * r + sub // 2) * 512
                    + yf * 32 + (2 * cc + sub % 2))

    i32s_ref[0] = b * 12288 + ch * 4096 + r * 256 + y * 16 + cc

    lmask = lax.broadcasted_iota(jnp.int32, (1, 512), 1)
    om_ref[0] = jnp.where(lmask == 0, -1.0,
                          jnp.where(lmask <= _N16, 1.0, 2.0))


def _select_call(imp):
    return pl.pallas_call(
        _select_body,
        grid=(_B,),
        in_specs=[
            pl.BlockSpec((1, 1, _NC), lambda b: (b, 0, 0)),
            pl.BlockSpec((1, _NC, 1), lambda b: (b, 0, 0)),
        ],
        out_specs=[
            pl.BlockSpec((1, _N16, 48), lambda b: (b, 0, 0)),
            pl.BlockSpec((1, _N32, 192), lambda b: (b, 0, 0)),
            pl.BlockSpec((1, _N32, 48), lambda b: (b, 0, 0)),
            pl.BlockSpec((1, 1, 512), lambda b: (b, 0, 0)),
        ],
        out_shape=[
            jax.ShapeDtypeStruct((_B, _N16, 48), jnp.int32),
            jax.ShapeDtypeStruct((_B, _N32, 192), jnp.int32),
            jax.ShapeDtypeStruct((_B, _N32, 48), jnp.int32),
            jax.ShapeDtypeStruct((_B, 1, 512), jnp.float32),
        ],
    )(imp.reshape(_B, 1, _NC), imp.reshape(_B, _NC, 1))


def _resize_body(img_ref, small_ref):
    def tri(d):
        w = jnp.where((d == 0) | (d == 1), 3.0, 0.0)
        return w + jnp.where((d == -1) | (d == 2), 1.0, 0.0)

    io2 = lax.broadcasted_iota(jnp.int32, (256, 512), 0)
    jo2 = lax.broadcasted_iota(jnp.int32, (256, 512), 1)
    den_r = jnp.where((io2 == 0) | (io2 == 255), 7.0, 8.0)
    w_r = tri(jo2 - 2 * io2) / den_r
    io3 = lax.broadcasted_iota(jnp.int32, (512, 256), 1)
    jo3 = lax.broadcasted_iota(jnp.int32, (512, 256), 0)
    den_c = jnp.where((io3 == 0) | (io3 == 255), 7.0, 8.0)
    w_c = tri(jo3 - 2 * io3) / den_c
    for c in range(3):
        t = jnp.dot(w_r, img_ref[0, c], preferred_element_type=jnp.float32,
                    precision=_HP)
        small_ref[0, c] = jnp.dot(t, w_c, preferred_element_type=jnp.float32,
                                  precision=_HP)


def _resize_call(images):
    return pl.pallas_call(
        _resize_body,
        grid=(_B,),
        in_specs=[pl.BlockSpec((1, 3, _IMG, _IMG), lambda b: (b, 0, 0, 0))],
        out_specs=pl.BlockSpec((1, 3, 256, 256), lambda b: (b, 0, 0, 0)),
        out_shape=jax.ShapeDtypeStruct((_B, 3, 256, 256), jnp.float32),
    )(images)


def _sc_gather(img_t, small_t, i16, ifull, i32s):
    mesh = plsc.VectorSubcoreMesh(core_axis_name="c", subcore_axis_name="s")

    @functools.partial(
        pl.kernel,
        mesh=mesh,
        compiler_params=pltpu.CompilerParams(use_tc_tiling_on_sc=False),
        out_type=[
            jax.ShapeDtypeStruct((_SEG16_N, 16), jnp.float32),
            jax.ShapeDtypeStruct((_SEGF_N, 16), jnp.float32),
            jax.ShapeDtypeStruct((_SEG32_PAD, 16), jnp.float32),
        ],
        scratch_types=[
            pltpu.VMEM((1024,), jnp.int32),
            pltpu.VMEM((1024, 16), jnp.float32),
            pltpu.SemaphoreType.DMA,
        ],
    )
    def k(imgt, smallt, i16h, ifullh, i32h, o16, ofull, o32, idxv, buf, sem):
        wid = lax.axis_index("s") * 2 + lax.axis_index("c")

        def make_phase(idx_hbm, table, out_hbm, nsuper):
            def body(js, carry):
                sid = js * 32 + wid

                @pl.when(sid < nsuper)
                def _():
                    base = sid * 1024
                    pltpu.sync_copy(idx_hbm.at[pl.ds(base, 1024)], idxv)
                    copies = []
                    for j in range(8):
                        copies.append(pltpu.async_copy(
                            table.at[idxv.at[pl.ds(j * 128, 128)]],
                            buf.at[pl.ds(j * 128, 128)], sem))
                    for cp in copies:
                        cp.wait()
                    pltpu.sync_copy(buf, out_hbm.at[pl.ds(base, 1024)])

                return carry

            lax.fori_loop(0, (nsuper + 31) // 32, body, 0)

        make_phase(i16h, imgt, o16, _SEG16_N // 1024)
        make_phase(ifullh, imgt, ofull, _SEGF_N // 1024)
        make_phase(i32h, smallt, o32, _SEG32_PAD // 1024)

    return k(img_t, small_t, i16, ifull, i32s)


def kernel(images):
    imgs01 = (images * jnp.asarray(_STD).reshape(1, 3, 1, 1)
              + jnp.asarray(_MEAN).reshape(1, 3, 1, 1))
    gray = imgs01.mean(axis=1)
    bins = jnp.clip((gray * _NBINS).astype(jnp.int32), 0, _NBINS - 1)

    hist = _hist_call(bins)

    p = hist / float(32 * 32)
    ent = -jnp.sum(jnp.where(p > 0, p * jnp.log2(jnp.maximum(p, 1e-12)), 0.0),
                   axis=-1)
    imp = ent.reshape(_B, _NC)

    i16, ifull, i32s, om = _select_call(imp)
    small = _resize_call(images)

    i32s_flat = i32s.reshape(-1)
    i32s_pad = jnp.concatenate(
        [i32s_flat, jnp.zeros((_SEG32_PAD - _SEG32_N,), jnp.int32)])
    o16, ofull, o32 = _sc_gather(
        images.reshape(-1, 16), small.reshape(-1, 16),
        i16.reshape(-1), ifull.reshape(-1), i32s_pad)

    resized16 = o16.reshape(-1, 3, _BASE, _BASE)
    full32 = ofull.reshape(-1, 4, 3, _BASE, _BASE)
    resized32 = o32[:_SEG32_N].reshape(-1, 3, _BASE, _BASE)
    output_mask = om.reshape(-1)
    seqlens = jnp.full((_B,), 1 + _N16 + _N32, jnp.int32)
    return (resized16, resized32, full32, output_mask, seqlens)
